# baseline (device time: 55212 ns/iter reference)
import jax
import jax.numpy as jnp
from jax import lax
from jax.experimental import pallas as pl
from jax.experimental.pallas import tpu as pltpu

HALF = 2048
D = 2048
QROWS = 512
K = 8
CH = QROWS // K
N_FLOWS = 6
X_CHUNKS = (0, 1, 2)
Z_CHUNKS = (3, 4, 5)
YD_CHUNKS = (6, 7)
YD = len(YD_CHUNKS)


def kernel(partial, gamma):
    gamma2 = gamma.reshape(1, D)

    def body(
        partial_ref,
        gamma_ref,
        out_ref,
        mine_ref,
        stage_ref,
        sendq_ref,
        remote_ref,
        ostage_ref,
        ssems,
        rsems,
        dsems,
        osems,
    ):
        mx = lax.axis_index("x")
        my = lax.axis_index("y")
        mz = lax.axis_index("z")
        q_own = 2 * mx + mz
        q_z = 2 * mx + (1 - mz)
        q_x = 2 * (1 - mx) + mz
        q_d = 2 * (1 - mx) + (1 - mz)
        nbr_y = (mx, 1 - my, mz)
        nbr_z = (mx, my, 1 - mz)
        nbr_x = (1 - mx, my, mz)

        barrier = pltpu.get_barrier_semaphore()
        for nbr in (nbr_y, nbr_z, nbr_x):
            pl.semaphore_signal(
                barrier, inc=1, device_id=nbr, device_id_type=pl.DeviceIdType.MESH
            )
        pl.semaphore_wait(barrier, 3)

        other_base = (1 - my) * HALF
        stage_dmas = []
        for c in range(K):
            d = pltpu.make_async_copy(
                partial_ref.at[0, pl.ds(other_base + q_own * QROWS + c * CH, CH), :],
                stage_ref.at[pl.ds(c * CH, CH)],
                dsems.at[1 + c],
            )
            d.start()
            stage_dmas.append(d)
        for i, c in enumerate(YD_CHUNKS):
            d = pltpu.make_async_copy(
                partial_ref.at[0, pl.ds(other_base + q_d * QROWS + c * CH, CH), :],
                stage_ref.at[pl.ds(QROWS + i * CH, CH)],
                dsems.at[1 + K + i],
            )
            d.start()
            stage_dmas.append(d)
        mine_dma = pltpu.make_async_copy(
            partial_ref.at[0, pl.ds(my * HALF, HALF), :], mine_ref, dsems.at[0]
        )
        mine_dma.start()

        def copy(src, dst, flow, c, dev):
            return pltpu.make_async_remote_copy(
                src_ref=src,
                dst_ref=dst,
                send_sem=ssems.at[flow, c],
                recv_sem=rsems.at[flow, c],
                device_id=dev,
                device_id_type=pl.DeviceIdType.MESH,
            )

        y_rdmas = []
        for c in range(K):
            ch = pl.ds(c * CH, CH)
            stage_dmas[c].wait()
            sendq_ref[ch, :] = stage_ref[ch, :].astype(jnp.bfloat16)
            r = copy(
                sendq_ref.at[ch],
                remote_ref.at[pl.ds(q_own * QROWS + c * CH, CH)],
                0,
                c,
                nbr_y,
            )
            r.start()
            y_rdmas.append(r)
        yd_rdmas = []
        for i, c in enumerate(YD_CHUNKS):
            ch = pl.ds(QROWS + i * CH, CH)
            stage_dmas[K + i].wait()
            sendq_ref[ch, :] = stage_ref[ch, :].astype(jnp.bfloat16)
            r = copy(
                sendq_ref.at[ch],
                remote_ref.at[pl.ds(q_d * QROWS + c * CH, CH)],
                5,
                i,
                nbr_y,
            )
            r.start()
            yd_rdmas.append(r)

        def compute_rows(rows):
            yv = mine_ref[rows, :] + remote_ref[rows, :].astype(jnp.float32)
            rms = jnp.sqrt(jnp.mean(yv * yv, axis=-1, keepdims=True) + 1e-6)
            ostage_ref[rows, :] = (yv / rms * gamma_ref[...]).astype(jnp.bfloat16)

        out_dmas = []

        def flush_quarter(qidx, osem):
            rows = pl.ds(qidx * QROWS, QROWS)
            d = pltpu.make_async_copy(ostage_ref.at[rows], out_ref.at[rows], osem)
            d.start()
            out_dmas.append(d)

        def recv_desc(rows, flow, c, dev):
            return copy(remote_ref.at[rows], remote_ref.at[rows], flow, c, dev)

        mine_dma.wait()
        z_fwds, x_fwds, d_fwds = [], [], []
        for c in range(K):
            rows = pl.ds(q_own * QROWS + c * CH, CH)
            y_rdmas[c].wait_recv()
            rz = copy(remote_ref.at[rows], remote_ref.at[rows], 1, c, nbr_z)
            rz.start()
            z_fwds.append(rz)
            rx = copy(remote_ref.at[rows], remote_ref.at[rows], 2, c, nbr_x)
            rx.start()
            x_fwds.append(rx)
            compute_rows(rows)
        flush_quarter(q_own, osems.at[0])

        for c in range(K):
            rows_z = pl.ds(q_z * QROWS + c * CH, CH)
            recv_desc(rows_z, 1, c, nbr_z).wait_recv()
            if c in X_CHUNKS:
                rdx = copy(remote_ref.at[rows_z], remote_ref.at[rows_z], 3, c, nbr_x)
                rdx.start()
                d_fwds.append(rdx)

            rows_x = pl.ds(q_x * QROWS + c * CH, CH)
            recv_desc(rows_x, 2, c, nbr_x).wait_recv()
            if c in Z_CHUNKS:
                rdz = copy(remote_ref.at[rows_x], remote_ref.at[rows_x], 4, c, nbr_z)
                rdz.start()
                d_fwds.append(rdz)

            compute_rows(rows_z)
            compute_rows(rows_x)

        flush_quarter(q_z, osems.at[1])
        flush_quarter(q_x, osems.at[2])

        for c in X_CHUNKS:
            rows = pl.ds(q_d * QROWS + c * CH, CH)
            recv_desc(rows, 3, c, nbr_x).wait_recv()
            compute_rows(rows)
        for c in Z_CHUNKS:
            rows = pl.ds(q_d * QROWS + c * CH, CH)
            recv_desc(rows, 4, c, nbr_z).wait_recv()
            compute_rows(rows)
        for i, c in enumerate(YD_CHUNKS):
            yd_rdmas[i].wait_recv()
            compute_rows(pl.ds(q_d * QROWS + c * CH, CH))
        flush_quarter(q_d, osems.at[3])

        for r in y_rdmas + yd_rdmas + z_fwds + x_fwds + d_fwds:
            r.wait_send()
        for d in out_dmas:
            d.wait()

    return pl.pallas_call(
        body,
        out_shape=jax.ShapeDtypeStruct((HALF, D), jnp.bfloat16),
        in_specs=[
            pl.BlockSpec(memory_space=pl.ANY),
            pl.BlockSpec(memory_space=pltpu.VMEM),
        ],
        out_specs=pl.BlockSpec(memory_space=pl.ANY),
        scratch_shapes=[
            pltpu.VMEM((HALF, D), jnp.float32),
            pltpu.VMEM((QROWS + YD * CH, D), jnp.float32),
            pltpu.VMEM((QROWS + YD * CH, D), jnp.bfloat16),
            pltpu.VMEM((HALF, D), jnp.bfloat16),
            pltpu.VMEM((HALF, D), jnp.bfloat16),
            pltpu.SemaphoreType.DMA((N_FLOWS, K)),
            pltpu.SemaphoreType.DMA((N_FLOWS, K)),
            pltpu.SemaphoreType.DMA((1 + K + YD,)),
            pltpu.SemaphoreType.DMA((4,)),
        ],
        compiler_params=pltpu.CompilerParams(
            collective_id=0, vmem_limit_bytes=100 * 1024 * 1024
        ),
    )(partial, gamma2)


# device time: 53891 ns/iter; 1.0245x vs baseline; 1.0245x over previous
import jax
import jax.numpy as jnp
from jax import lax
from jax.experimental import pallas as pl
from jax.experimental.pallas import tpu as pltpu

HALF = 2048
D = 2048
QROWS = 512
K = 8
CH = QROWS // K
N_FLOWS = 6
X_CHUNKS = (0, 1, 2)
Z_CHUNKS = (3, 4, 5)
YD_CHUNKS = (6, 7)
YD = len(YD_CHUNKS)


def kernel(partial, gamma):
    gamma2 = gamma.reshape(1, D)

    def body(
        partial_ref,
        gamma_ref,
        out_ref,
        mine_ref,
        stage_ref,
        sendq_ref,
        remote_ref,
        ostage_ref,
        ssems,
        rsems,
        dsems,
        osems,
    ):
        mx = lax.axis_index("x")
        my = lax.axis_index("y")
        mz = lax.axis_index("z")
        q_own = 2 * mx + mz
        q_z = 2 * mx + (1 - mz)
        q_x = 2 * (1 - mx) + mz
        q_d = 2 * (1 - mx) + (1 - mz)
        nbr_y = (mx, 1 - my, mz)
        nbr_z = (mx, my, 1 - mz)
        nbr_x = (1 - mx, my, mz)

        other_base = (1 - my) * HALF
        stage_dmas = []
        for c in range(K):
            d = pltpu.make_async_copy(
                partial_ref.at[0, pl.ds(other_base + q_own * QROWS + c * CH, CH), :],
                stage_ref.at[pl.ds(c * CH, CH)],
                dsems.at[1 + c],
            )
            d.start()
            stage_dmas.append(d)
        for i, c in enumerate(YD_CHUNKS):
            d = pltpu.make_async_copy(
                partial_ref.at[0, pl.ds(other_base + q_d * QROWS + c * CH, CH), :],
                stage_ref.at[pl.ds(QROWS + i * CH, CH)],
                dsems.at[1 + K + i],
            )
            d.start()
            stage_dmas.append(d)
        mine_dma = pltpu.make_async_copy(
            partial_ref.at[0, pl.ds(my * HALF, HALF), :], mine_ref, dsems.at[0]
        )
        mine_dma.start()

        barrier = pltpu.get_barrier_semaphore()
        for nbr in (nbr_y, nbr_z, nbr_x):
            pl.semaphore_signal(
                barrier, inc=1, device_id=nbr, device_id_type=pl.DeviceIdType.MESH
            )
        pl.semaphore_wait(barrier, 3)

        def copy(src, dst, flow, c, dev):
            return pltpu.make_async_remote_copy(
                src_ref=src,
                dst_ref=dst,
                send_sem=ssems.at[flow, c],
                recv_sem=rsems.at[flow, c],
                device_id=dev,
                device_id_type=pl.DeviceIdType.MESH,
            )

        y_rdmas = []
        for c in range(K):
            ch = pl.ds(c * CH, CH)
            stage_dmas[c].wait()
            sendq_ref[ch, :] = stage_ref[ch, :].astype(jnp.bfloat16)
            r = copy(
                sendq_ref.at[ch],
                remote_ref.at[pl.ds(q_own * QROWS + c * CH, CH)],
                0,
                c,
                nbr_y,
            )
            r.start()
            y_rdmas.append(r)
        yd_rdmas = []
        for i, c in enumerate(YD_CHUNKS):
            ch = pl.ds(QROWS + i * CH, CH)
            stage_dmas[K + i].wait()
            sendq_ref[ch, :] = stage_ref[ch, :].astype(jnp.bfloat16)
            r = copy(
                sendq_ref.at[ch],
                remote_ref.at[pl.ds(q_d * QROWS + c * CH, CH)],
                5,
                i,
                nbr_y,
            )
            r.start()
            yd_rdmas.append(r)

        def compute_rows(rows):
            yv = mine_ref[rows, :] + remote_ref[rows, :].astype(jnp.float32)
            inv = lax.rsqrt(jnp.mean(yv * yv, axis=-1, keepdims=True) + 1e-6)
            ostage_ref[rows, :] = (yv * inv * gamma_ref[...]).astype(jnp.bfloat16)

        out_dmas = []

        def flush_quarter(qidx, osem):
            rows = pl.ds(qidx * QROWS, QROWS)
            d = pltpu.make_async_copy(ostage_ref.at[rows], out_ref.at[rows], osem)
            d.start()
            out_dmas.append(d)

        def recv_desc(rows, flow, c, dev):
            return copy(remote_ref.at[rows], remote_ref.at[rows], flow, c, dev)

        mine_dma.wait()
        z_fwds, x_fwds, d_fwds = [], [], []
        for c in range(K):
            rows = pl.ds(q_own * QROWS + c * CH, CH)
            y_rdmas[c].wait_recv()
            rz = copy(remote_ref.at[rows], remote_ref.at[rows], 1, c, nbr_z)
            rz.start()
            z_fwds.append(rz)
            rx = copy(remote_ref.at[rows], remote_ref.at[rows], 2, c, nbr_x)
            rx.start()
            x_fwds.append(rx)
            compute_rows(rows)
        flush_quarter(q_own, osems.at[0])

        for c in range(K):
            rows_z = pl.ds(q_z * QROWS + c * CH, CH)
            recv_desc(rows_z, 1, c, nbr_z).wait_recv()
            if c in X_CHUNKS:
                rdx = copy(remote_ref.at[rows_z], remote_ref.at[rows_z], 3, c, nbr_x)
                rdx.start()
                d_fwds.append(rdx)

            rows_x = pl.ds(q_x * QROWS + c * CH, CH)
            recv_desc(rows_x, 2, c, nbr_x).wait_recv()
            if c in Z_CHUNKS:
                rdz = copy(remote_ref.at[rows_x], remote_ref.at[rows_x], 4, c, nbr_z)
                rdz.start()
                d_fwds.append(rdz)

            compute_rows(rows_z)
            compute_rows(rows_x)

        flush_quarter(q_z, osems.at[1])
        flush_quarter(q_x, osems.at[2])

        for c in X_CHUNKS:
            rows = pl.ds(q_d * QROWS + c * CH, CH)
            recv_desc(rows, 3, c, nbr_x).wait_recv()
            compute_rows(rows)
        for c in Z_CHUNKS:
            rows = pl.ds(q_d * QROWS + c * CH, CH)
            recv_desc(rows, 4, c, nbr_z).wait_recv()
            compute_rows(rows)
        for i, c in enumerate(YD_CHUNKS):
            yd_rdmas[i].wait_recv()
            compute_rows(pl.ds(q_d * QROWS + c * CH, CH))
        flush_quarter(q_d, osems.at[3])

        for r in y_rdmas + yd_rdmas + z_fwds + x_fwds + d_fwds:
            r.wait_send()
        for d in out_dmas:
            d.wait()

    return pl.pallas_call(
        body,
        out_shape=jax.ShapeDtypeStruct((HALF, D), jnp.bfloat16),
        in_specs=[
            pl.BlockSpec(memory_space=pl.ANY),
            pl.BlockSpec(memory_space=pltpu.VMEM),
        ],
        out_specs=pl.BlockSpec(memory_space=pl.ANY),
        scratch_shapes=[
            pltpu.VMEM((HALF, D), jnp.float32),
            pltpu.VMEM((QROWS + YD * CH, D), jnp.float32),
            pltpu.VMEM((QROWS + YD * CH, D), jnp.bfloat16),
            pltpu.VMEM((HALF, D), jnp.bfloat16),
            pltpu.VMEM((HALF, D), jnp.bfloat16),
            pltpu.SemaphoreType.DMA((N_FLOWS, K)),
            pltpu.SemaphoreType.DMA((N_FLOWS, K)),
            pltpu.SemaphoreType.DMA((1 + K + YD,)),
            pltpu.SemaphoreType.DMA((4,)),
        ],
        compiler_params=pltpu.CompilerParams(
            collective_id=0, vmem_limit_bytes=100 * 1024 * 1024
        ),
    )(partial, gamma2)
